# baseline (speedup 1.0000x reference)
import functools

import jax
import jax.numpy as jnp
from jax import lax
from jax.experimental import pallas as pl
from jax.experimental.pallas import tpu as pltpu
from jax.experimental.pallas import tpu_sc as plsc

VOCAB = 1000000
DIM = 64
B = 4096
L = 50

NC, NS = 2, 16
NW = NC * NS
TOTAL = B * L
PER_W = TOTAL // NW
N_CHUNK = 8
CH = PER_W // N_CHUNK


def _body(idx_hbm, table_hbm, out_hbm, idx_v, bufs, gsem, wsem):
    wid = lax.axis_index("s") * NC + lax.axis_index("c")
    base = wid * PER_W
    pltpu.sync_copy(idx_hbm.at[pl.ds(base, PER_W)], idx_v)

    gd = {}
    wd = {}
    for c in range(N_CHUNK):
        b = c % 2
        if c == 0:
            gd[0] = pltpu.async_copy(
                table_hbm.at[idx_v.at[pl.ds(0, CH)]], bufs.at[0], gsem.at[0])
        if c + 1 < N_CHUNK:
            nb = (c + 1) % 2
            if c - 1 >= 0:
                wd[c - 1].wait()
            gd[c + 1] = pltpu.async_copy(
                table_hbm.at[idx_v.at[pl.ds((c + 1) * CH, CH)]],
                bufs.at[nb], gsem.at[nb])
        gd[c].wait()
        wd[c] = pltpu.async_copy(
            bufs.at[b], out_hbm.at[pl.ds(base + c * CH, CH)], wsem.at[b])
    wd[N_CHUNK - 2].wait()
    wd[N_CHUNK - 1].wait()


@jax.jit
def _embed(x_flat, E):
    mesh = plsc.VectorSubcoreMesh(core_axis_name="c", subcore_axis_name="s")
    return pl.kernel(
        _body,
        out_type=jax.ShapeDtypeStruct((TOTAL, DIM), jnp.float32),
        mesh=mesh,
        scratch_types=[
            pltpu.VMEM((PER_W,), jnp.int32),
            pltpu.VMEM((2, CH, DIM), jnp.float32),
            pltpu.SemaphoreType.DMA((2,)),
            pltpu.SemaphoreType.DMA((2,)),
        ],
        compiler_params=pltpu.CompilerParams(use_tc_tiling_on_sc=False),
    )(x_flat, E)


def kernel(x, E):
    flat = x.reshape(-1).astype(jnp.int32)
    out = _embed(flat, E)
    return out.reshape(x.shape[0], x.shape[1], E.shape[1])
